# baseline (speedup 1.0000x reference)
import functools

import jax
import jax.numpy as jnp
from jax import lax
from jax.experimental import pallas as pl
from jax.experimental.pallas import tpu as pltpu
from jax.experimental.pallas import tpu_sc as plsc

_VE_DIM = 64
_MODEL_DIM = 128
_CHUNK = 128
_NUM_WORKERS = 32
_SEQ_BLOCK = 512


def _sc_gather(table, ids_2d, n_rows):
    chunks_per_worker = ids_2d.shape[0] // _NUM_WORKERS
    rows_per_worker = chunks_per_worker * _CHUNK
    mesh = plsc.VectorSubcoreMesh(core_axis_name="c", subcore_axis_name="s")
    cp = pltpu.CompilerParams(use_tc_tiling_on_sc=False)

    @functools.partial(
        pl.kernel,
        mesh=mesh,
        out_type=jax.ShapeDtypeStruct((n_rows, _VE_DIM), table.dtype),
        scratch_types=[
            pltpu.VMEM((chunks_per_worker, _CHUNK), jnp.int32),
            pltpu.VMEM((_CHUNK, _VE_DIM), jnp.float32),
            pltpu.VMEM((_CHUNK, _VE_DIM), jnp.float32),
            pltpu.SemaphoreType.DMA,
            pltpu.SemaphoreType.DMA,
        ],
        compiler_params=cp,
    )
    def gather_kernel(table_hbm, ids_hbm, out_hbm, idx_v, buf0, buf1, sem0, sem1):
        wid = lax.axis_index("s") * 2 + lax.axis_index("c")
        base = wid * rows_per_worker
        pltpu.sync_copy(
            ids_hbm.at[pl.ds(wid * chunks_per_worker, chunks_per_worker)],
            idx_v,
        )

        pltpu.make_async_copy(table_hbm.at[idx_v.at[0]], buf0, sem0).start()

        @pl.loop(0, chunks_per_worker // 2)
        def _(g):
            j0 = 2 * g
            pltpu.make_async_copy(table_hbm.at[idx_v.at[j0]], buf0, sem0).wait()
            pltpu.make_async_copy(
                table_hbm.at[idx_v.at[j0 + 1]], buf1, sem1
            ).start()
            pltpu.sync_copy(buf0, out_hbm.at[pl.ds(base + j0 * _CHUNK, _CHUNK)])
            pltpu.make_async_copy(
                table_hbm.at[idx_v.at[j0 + 1]], buf1, sem1
            ).wait()

            @pl.when(g + 1 < chunks_per_worker // 2)
            def _():
                pltpu.make_async_copy(
                    table_hbm.at[idx_v.at[j0 + 2]], buf0, sem0
                ).start()

            pltpu.sync_copy(
                buf1, out_hbm.at[pl.ds(base + (j0 + 1) * _CHUNK, _CHUNK)]
            )

    return gather_kernel(table, ids_2d)


def _proj_body(x_hbm, w_ref, o_ref, xbuf, sem):
    pair_blk = xbuf.shape[0]
    i = pl.program_id(0)
    pltpu.make_async_copy(
        x_hbm.at[pl.ds(i * pair_blk, pair_blk)], xbuf, sem
    ).start()
    pltpu.make_async_copy(
        x_hbm.at[pl.ds(i * pair_blk, pair_blk)], xbuf, sem
    ).wait()
    y = jnp.dot(xbuf[...], w_ref[...], preferred_element_type=jnp.float32)
    o_ref[...] = y.reshape(o_ref.shape)


def _tc_project(pairs2, w2b, batch, seq):
    pair_blk = _SEQ_BLOCK * seq // 2
    return pl.pallas_call(
        _proj_body,
        grid=(batch // _SEQ_BLOCK,),
        in_specs=[
            pl.BlockSpec(memory_space=pltpu.MemorySpace.HBM),
            pl.BlockSpec((2 * _VE_DIM, 2 * _MODEL_DIM), lambda i: (0, 0)),
        ],
        out_specs=pl.BlockSpec(
            (_SEQ_BLOCK, seq, _MODEL_DIM), lambda i: (i, 0, 0)
        ),
        out_shape=jax.ShapeDtypeStruct((batch, seq, _MODEL_DIM), jnp.float32),
        scratch_shapes=[
            pltpu.VMEM((pair_blk, 2 * _VE_DIM), jnp.float32),
            pltpu.SemaphoreType.DMA,
        ],
    )(pairs2, w2b)


def kernel(token_ids, embed_weight, proj_weight, scale):
    batch, seq = token_ids.shape
    n_rows = batch * seq
    ids_2d = token_ids.reshape(n_rows // _CHUNK, _CHUNK).astype(jnp.int32)
    w = proj_weight.astype(jnp.float32).T * scale.astype(jnp.float32)
    w2b = jnp.zeros((2 * _VE_DIM, 2 * _MODEL_DIM), jnp.float32)
    w2b = w2b.at[:_VE_DIM, :_MODEL_DIM].set(w)
    w2b = w2b.at[_VE_DIM:, _MODEL_DIM:].set(w)
    gathered = _sc_gather(embed_weight, ids_2d, n_rows)
    pairs2 = gathered.reshape(n_rows // 2, 2 * _VE_DIM)
    return _tc_project(pairs2, w2b, batch, seq)


# speedup vs baseline: 1.0568x; 1.0568x over previous
import functools

import jax
import jax.numpy as jnp
from jax import lax
from jax.experimental import pallas as pl
from jax.experimental.pallas import tpu as pltpu
from jax.experimental.pallas import tpu_sc as plsc

_VE_DIM = 64
_MODEL_DIM = 128
_CHUNK = 128
_NUM_WORKERS = 32
_SEQ_BLOCK = 512


def _sc_gather(table, ids_2d, n_rows):
    chunks_per_worker = ids_2d.shape[0] // _NUM_WORKERS
    rows_per_worker = chunks_per_worker * _CHUNK
    mesh = plsc.VectorSubcoreMesh(core_axis_name="c", subcore_axis_name="s")
    cp = pltpu.CompilerParams(use_tc_tiling_on_sc=False)

    @functools.partial(
        pl.kernel,
        mesh=mesh,
        out_type=jax.ShapeDtypeStruct((n_rows, _VE_DIM), table.dtype),
        scratch_types=[
            pltpu.VMEM((chunks_per_worker, _CHUNK), jnp.int32),
            pltpu.VMEM((_CHUNK, _VE_DIM), jnp.float32),
            pltpu.VMEM((_CHUNK, _VE_DIM), jnp.float32),
            pltpu.SemaphoreType.DMA,
            pltpu.SemaphoreType.DMA,
        ],
        compiler_params=cp,
    )
    def gather_kernel(table_hbm, ids_hbm, out_hbm, idx_v, buf0, buf1, sem0, sem1):
        wid = lax.axis_index("s") * 2 + lax.axis_index("c")
        base = wid * rows_per_worker
        pltpu.sync_copy(
            ids_hbm.at[pl.ds(wid * chunks_per_worker, chunks_per_worker)],
            idx_v,
        )

        pltpu.make_async_copy(table_hbm.at[idx_v.at[0]], buf0, sem0).start()

        @pl.loop(0, chunks_per_worker // 2)
        def _(g):
            j0 = 2 * g
            pltpu.make_async_copy(table_hbm.at[idx_v.at[j0]], buf0, sem0).wait()
            pltpu.make_async_copy(
                table_hbm.at[idx_v.at[j0 + 1]], buf1, sem1
            ).start()
            pltpu.sync_copy(buf0, out_hbm.at[pl.ds(base + j0 * _CHUNK, _CHUNK)])
            pltpu.make_async_copy(
                table_hbm.at[idx_v.at[j0 + 1]], buf1, sem1
            ).wait()

            @pl.when(g + 1 < chunks_per_worker // 2)
            def _():
                pltpu.make_async_copy(
                    table_hbm.at[idx_v.at[j0 + 2]], buf0, sem0
                ).start()

            pltpu.sync_copy(
                buf1, out_hbm.at[pl.ds(base + (j0 + 1) * _CHUNK, _CHUNK)]
            )

    return gather_kernel(table, ids_2d)


def _proj_body(x_ref, w_ref, o_ref):
    y = jnp.dot(x_ref[...], w_ref[...], preferred_element_type=jnp.float32)
    o_ref[...] = y.reshape(o_ref.shape)


def _tc_project(pairs2, w2b, batch, seq):
    pair_blk = _SEQ_BLOCK * seq // 2
    return pl.pallas_call(
        _proj_body,
        grid=(batch // _SEQ_BLOCK,),
        in_specs=[
            pl.BlockSpec((pair_blk, 2 * _VE_DIM), lambda i: (i, 0)),
            pl.BlockSpec((2 * _VE_DIM, 2 * _MODEL_DIM), lambda i: (0, 0)),
        ],
        out_specs=pl.BlockSpec(
            (_SEQ_BLOCK, seq, _MODEL_DIM), lambda i: (i, 0, 0)
        ),
        out_shape=jax.ShapeDtypeStruct((batch, seq, _MODEL_DIM), jnp.float32),
    )(pairs2, w2b)


def kernel(token_ids, embed_weight, proj_weight, scale):
    batch, seq = token_ids.shape
    n_rows = batch * seq
    ids_2d = token_ids.reshape(n_rows // _CHUNK, _CHUNK).astype(jnp.int32)
    w = proj_weight.astype(jnp.float32).T * scale.astype(jnp.float32)
    w2b = jnp.zeros((2 * _VE_DIM, 2 * _MODEL_DIM), jnp.float32)
    w2b = w2b.at[:_VE_DIM, :_MODEL_DIM].set(w)
    w2b = w2b.at[_VE_DIM:, _MODEL_DIM:].set(w)
    gathered = _sc_gather(embed_weight, ids_2d, n_rows)
    pairs2 = gathered.reshape(n_rows // 2, 2 * _VE_DIM)
    return _tc_project(pairs2, w2b, batch, seq)
